# Initial kernel scaffold; baseline (speedup 1.0000x reference)
import jax
import jax.numpy as jnp
from jax.experimental import pallas as pl


def kernel(x, q_Y_sample, adj, t, num_steps, W_t1, b_t1, W_t2, b_t2, Wl0, bl0, Wr0, Wl1, bl1, Wr1, Wf1, bf1, Wf2, bf2):
    raise NotImplementedError("write your pallas kernel here")


# speedup vs baseline: 5.4937x; 5.4937x over previous
import math

import jax
import jax.numpy as jnp
from jax import lax
from jax.experimental import pallas as pl
from jax.experimental.pallas import tpu as pltpu
from jax.experimental.pallas import tpu_sc as plsc

N = 10000
E = 320000
NHID = 64

NC = 2
NS = 16
NW = NC * NS
EPT = E // NW
CH = 80
NITER = EPT // CH
RPT = 632
NP = NS * RPT

_HIGH = jax.lax.Precision.HIGHEST


def _dot(a, b):
  return jax.lax.dot(a, b, precision=_HIGH)


def _elu(v):
  return jnp.where(v > 0, v, jnp.exp(jnp.minimum(v, 0.0)) - 1.0)


def _make_sc_agg(d):
  mesh = plsc.VectorSubcoreMesh(
      core_axis_name="c", subcore_axis_name="s", num_cores=NC, num_subcores=NS)

  def body(table, src_idx, dst_idx, out, src_v, dst_v, rows_v, zbuf, acc, sem):
    c = lax.axis_index("c")
    s = lax.axis_index("s")
    wid = c * NS + s

    zeros = jnp.zeros((16,), jnp.float32)

    def zinit(r, carry):
      for k in range(d // 16):
        zbuf[r, pl.ds(k * 16, 16)] = zeros
      return carry

    lax.fori_loop(0, RPT, zinit, 0)
    pltpu.sync_copy(zbuf, acc.at[pl.ds(s * RPT, RPT)])
    plsc.subcore_barrier()

    e0 = wid * EPT

    def eloop(i, carry):
      off = e0 + i * CH
      pltpu.sync_copy(src_idx.at[pl.ds(off, CH)], src_v)
      pltpu.sync_copy(dst_idx.at[pl.ds(off, CH)], dst_v)
      pltpu.async_copy(table.at[src_v], rows_v, sem).wait()
      pltpu.sync_copy(rows_v, acc.at[dst_v], add=True)
      return carry

    lax.fori_loop(0, NITER, eloop, 0)
    plsc.subcore_barrier()

    pltpu.sync_copy(acc.at[pl.ds(s * RPT, RPT)], out.at[c, pl.ds(s * RPT, RPT)])

  return pl.kernel(
      body,
      out_type=jax.ShapeDtypeStruct((NC, NP, d), jnp.float32),
      mesh=mesh,
      scratch_types=[
          pltpu.VMEM((CH,), jnp.int32),
          pltpu.VMEM((CH,), jnp.int32),
          pltpu.VMEM((CH, d), jnp.float32),
          pltpu.VMEM((RPT, d), jnp.float32),
          pltpu.VMEM_SHARED((NP, d), jnp.float32),
          pltpu.SemaphoreType.DMA,
      ],
      compiler_params=pltpu.CompilerParams(use_tc_tiling_on_sc=False),
  )


_sc_agg80 = _make_sc_agg(80)
_sc_agg64 = _make_sc_agg(64)


RB = 2000
GRID = N // RB


def _ka_body(t_ref, x_ref, q_ref, freq_ref, Wt1, bt1, Wt2, bt2,
             Wl0x, Wl0q, Wr0x, Wr0q, T0_ref, R0_ref, temb_ref):
  emb = (t_ref[...] * 4.0) * freq_ref[...]
  temb0 = jnp.concatenate([jnp.sin(emb), jnp.cos(emb)], axis=1)
  hmid = _elu(_dot(temb0, Wt1[...]) + bt1[...])
  temb_ref[...] = _dot(hmid, Wt2[...]) + bt2[...]
  x = x_ref[...]
  q = q_ref[...]
  g = _dot(x, Wl0x[...]) + _dot(q, Wl0q[...])
  T0_ref[...] = jnp.concatenate(
      [g, jnp.ones((RB, 1), jnp.float32), jnp.zeros((RB, 15), jnp.float32)],
      axis=1)
  R0_ref[...] = _dot(x, Wr0x[...]) + _dot(q, Wr0q[...])


def _kb_body(agg_ref, R0_ref, temb_ref, q_ref, bl0,
             Wl1x, Wl1q, Wr1x, Wr1q, T1_ref, R1_ref, rdeg_ref):
  a = agg_ref[0] + agg_ref[1]
  rdeg = 1.0 / jnp.maximum(a[:, 64:65], 1.0)
  out0 = a[:, :64] * rdeg + bl0[...] + R0_ref[...]
  nrm = jnp.sqrt(jnp.sum(out0 * out0, axis=1, keepdims=True))
  out0 = out0 / jnp.maximum(nrm, 1e-12)
  h = jnp.maximum(out0 + temb_ref[...], 0.0)
  q = q_ref[...]
  T1_ref[...] = _dot(h, Wl1x[...]) + _dot(q, Wl1q[...])
  R1_ref[...] = _dot(h, Wr1x[...]) + _dot(q, Wr1q[...])
  rdeg_ref[...] = rdeg


def _kc_body(agg_ref, rdeg_ref, R1_ref, temb_ref, q_ref, bl1,
             Wf1h, Wf1q, bf1, Wf2, bf2, out_ref):
  a = agg_ref[0] + agg_ref[1]
  out1 = a * rdeg_ref[...] + bl1[...] + R1_ref[...]
  nrm = jnp.sqrt(jnp.sum(out1 * out1, axis=1, keepdims=True))
  out1 = out1 / jnp.maximum(nrm, 1e-12)
  h = jnp.maximum(out1 + temb_ref[...], 0.0)
  q = q_ref[...]
  f = _elu(_dot(h, Wf1h[...]) + _dot(q, Wf1q[...]) + bf1[...])
  out_ref[...] = _dot(f, Wf2[...]) + bf2[...]


def _row_spec(d):
  return pl.BlockSpec((RB, d), lambda i: (i, 0))


def _full_spec(shape):
  nd = len(shape)
  return pl.BlockSpec(shape, lambda i, _n=nd: (0,) * _n)


def _agg_spec(d):
  return pl.BlockSpec((NC, RB, d), lambda i: (0, i, 0))


def kernel(x, q_Y_sample, adj, t, num_steps, W_t1, b_t1, W_t2, b_t2,
           Wl0, bl0, Wr0, Wl1, bl1, Wr1, Wf1, bf1, Wf2, bf2):
  del num_steps
  adj = adj.astype(jnp.int32)
  src = adj[0]
  dst = adj[1]
  f32 = jnp.float32
  half = NHID // 2
  freq = jnp.exp(
      jnp.arange(half, dtype=f32) * (-math.log(10000.0) / (half - 1)))
  freq = freq.reshape(1, half)

  nfeat = x.shape[1]
  q = q_Y_sample

  ka = pl.pallas_call(
      _ka_body,
      grid=(GRID,),
      in_specs=[
          _row_spec(1), _row_spec(nfeat), _row_spec(q.shape[1]),
          _full_spec(freq.shape),
          _full_spec(W_t1.shape), _full_spec((1, b_t1.shape[0])),
          _full_spec(W_t2.shape), _full_spec((1, b_t2.shape[0])),
          _full_spec((nfeat, NHID)), _full_spec((q.shape[1], NHID)),
          _full_spec((nfeat, NHID)), _full_spec((q.shape[1], NHID)),
      ],
      out_specs=[_row_spec(80), _row_spec(NHID), _row_spec(NHID)],
      out_shape=[
          jax.ShapeDtypeStruct((N, 80), f32),
          jax.ShapeDtypeStruct((N, NHID), f32),
          jax.ShapeDtypeStruct((N, NHID), f32),
      ],
  )
  T0, R0, temb = ka(
      t.reshape(N, 1), x, q, freq,
      W_t1, b_t1.reshape(1, -1), W_t2, b_t2.reshape(1, -1),
      Wl0[:nfeat], Wl0[nfeat:], Wr0[:nfeat], Wr0[nfeat:])

  agg0 = _sc_agg80(T0, src, dst)

  kb = pl.pallas_call(
      _kb_body,
      grid=(GRID,),
      in_specs=[
          _agg_spec(80), _row_spec(NHID), _row_spec(NHID),
          _row_spec(q.shape[1]), _full_spec((1, NHID)),
          _full_spec((NHID, NHID)), _full_spec((q.shape[1], NHID)),
          _full_spec((NHID, NHID)), _full_spec((q.shape[1], NHID)),
      ],
      out_specs=[_row_spec(NHID), _row_spec(NHID), _row_spec(1)],
      out_shape=[
          jax.ShapeDtypeStruct((N, NHID), f32),
          jax.ShapeDtypeStruct((N, NHID), f32),
          jax.ShapeDtypeStruct((N, 1), f32),
      ],
  )
  T1, R1, rdeg = kb(
      agg0, R0, temb, q, bl0.reshape(1, -1),
      Wl1[:NHID], Wl1[NHID:], Wr1[:NHID], Wr1[NHID:])

  agg1 = _sc_agg64(T1, src, dst)

  nout = bf2.shape[0]
  f1 = bf1.shape[0]
  kc = pl.pallas_call(
      _kc_body,
      grid=(GRID,),
      in_specs=[
          _agg_spec(NHID), _row_spec(1), _row_spec(NHID), _row_spec(NHID),
          _row_spec(q.shape[1]), _full_spec((1, NHID)),
          _full_spec((NHID, f1)), _full_spec((q.shape[1], f1)),
          _full_spec((1, f1)), _full_spec((f1, nout)), _full_spec((1, nout)),
      ],
      out_specs=[_row_spec(nout)],
      out_shape=[jax.ShapeDtypeStruct((N, nout), f32)],
  )
  (out,) = kc(
      agg1, rdeg, R1, temb, q, bl1.reshape(1, -1),
      Wf1[:NHID], Wf1[NHID:], bf1.reshape(1, -1), Wf2, bf2.reshape(1, -1))
  return out


# speedup vs baseline: 10.6028x; 1.9300x over previous
import math

import jax
import jax.numpy as jnp
from jax import lax
from jax.experimental import pallas as pl
from jax.experimental.pallas import tpu as pltpu
from jax.experimental.pallas import tpu_sc as plsc

N = 10000
E = 320000
NHID = 64

NC = 2
NS = 16
NW = NC * NS
EPT = E // NW
CH = 80
NITER = EPT // CH
RPT = 632
NP = NS * RPT

_HIGH = jax.lax.Precision.HIGHEST


def _dot(a, b):
  return jax.lax.dot(a, b, precision=_HIGH)


def _elu(v):
  return jnp.where(v > 0, v, jnp.exp(jnp.minimum(v, 0.0)) - 1.0)


ZR = RPT // 8


def _make_sc_agg(d):
  mesh = plsc.VectorSubcoreMesh(
      core_axis_name="c", subcore_axis_name="s", num_cores=NC, num_subcores=NS)

  def body(table, src2d, dst2d, out, srcv, dstv, rows, zbuf, acc, gsem, ssem):
    c = lax.axis_index("c")
    s = lax.axis_index("s")
    wid = c * NS + s

    zeros = jnp.zeros((16,), jnp.float32)

    def zinit(r, carry):
      for k in range(d // 16):
        zbuf[r, pl.ds(k * 16, 16)] = zeros
      return carry

    lax.fori_loop(0, ZR, zinit, 0)
    for z in range(RPT // ZR):
      pltpu.sync_copy(zbuf, acc.at[pl.ds(s * RPT + z * ZR, ZR)])
    plsc.subcore_barrier()

    pltpu.sync_copy(src2d.at[pl.ds(wid * NITER, NITER)], srcv)
    pltpu.sync_copy(dst2d.at[pl.ds(wid * NITER, NITER)], dstv)

    pltpu.async_copy(table.at[srcv.at[0]], rows.at[0], gsem.at[0])

    def eloop(i, carry):
      b = lax.rem(i, 2)
      nb = 1 - b

      @pl.when(i >= 1)
      def _():
        pltpu.make_async_copy(
            rows.at[nb], acc.at[dstv.at[i - 1]], ssem.at[nb]).wait()

      @pl.when(i + 1 < NITER)
      def _():
        pltpu.async_copy(table.at[srcv.at[i + 1]], rows.at[nb], gsem.at[nb])

      pltpu.make_async_copy(table.at[srcv.at[i]], rows.at[b], gsem.at[b]).wait()
      pltpu.async_copy(rows.at[b], acc.at[dstv.at[i]], ssem.at[b], add=True)
      return carry

    lax.fori_loop(0, NITER, eloop, 0)
    lb = (NITER - 1) % 2
    pltpu.make_async_copy(
        rows.at[lb], acc.at[dstv.at[NITER - 1]], ssem.at[lb]).wait()
    plsc.subcore_barrier()

    pltpu.sync_copy(acc.at[pl.ds(s * RPT, RPT)], out.at[c, pl.ds(s * RPT, RPT)])

  return pl.kernel(
      body,
      out_type=jax.ShapeDtypeStruct((NC, NP, d), jnp.float32),
      mesh=mesh,
      scratch_types=[
          pltpu.VMEM((NITER, CH), jnp.int32),
          pltpu.VMEM((NITER, CH), jnp.int32),
          pltpu.VMEM((2, CH, d), jnp.float32),
          pltpu.VMEM((ZR, d), jnp.float32),
          pltpu.VMEM_SHARED((NP, d), jnp.float32),
          pltpu.SemaphoreType.DMA((2,)),
          pltpu.SemaphoreType.DMA((2,)),
      ],
      compiler_params=pltpu.CompilerParams(use_tc_tiling_on_sc=False),
  )


_sc_agg80 = _make_sc_agg(80)
_sc_agg64 = _make_sc_agg(64)


RB = 2000
GRID = N // RB


def _ka_body(t_ref, x_ref, q_ref, freq_ref, Wt1, bt1, Wt2, bt2,
             Wl0x, Wl0q, Wr0x, Wr0q, T0_ref, R0_ref, temb_ref):
  emb = (t_ref[...] * 4.0) * freq_ref[...]
  temb0 = jnp.concatenate([jnp.sin(emb), jnp.cos(emb)], axis=1)
  hmid = _elu(_dot(temb0, Wt1[...]) + bt1[...])
  temb_ref[...] = _dot(hmid, Wt2[...]) + bt2[...]
  x = x_ref[...]
  q = q_ref[...]
  g = _dot(x, Wl0x[...]) + _dot(q, Wl0q[...])
  T0_ref[...] = jnp.concatenate(
      [g, jnp.ones((RB, 1), jnp.float32), jnp.zeros((RB, 15), jnp.float32)],
      axis=1)
  R0_ref[...] = _dot(x, Wr0x[...]) + _dot(q, Wr0q[...])


def _kb_body(agg_ref, R0_ref, temb_ref, q_ref, bl0,
             Wl1x, Wl1q, Wr1x, Wr1q, T1_ref, R1_ref, rdeg_ref):
  a = agg_ref[0] + agg_ref[1]
  rdeg = 1.0 / jnp.maximum(a[:, 64:65], 1.0)
  out0 = a[:, :64] * rdeg + bl0[...] + R0_ref[...]
  nrm = jnp.sqrt(jnp.sum(out0 * out0, axis=1, keepdims=True))
  out0 = out0 / jnp.maximum(nrm, 1e-12)
  h = jnp.maximum(out0 + temb_ref[...], 0.0)
  q = q_ref[...]
  T1_ref[...] = _dot(h, Wl1x[...]) + _dot(q, Wl1q[...])
  R1_ref[...] = _dot(h, Wr1x[...]) + _dot(q, Wr1q[...])
  rdeg_ref[...] = rdeg


def _kc_body(agg_ref, rdeg_ref, R1_ref, temb_ref, q_ref, bl1,
             Wf1h, Wf1q, bf1, Wf2, bf2, out_ref):
  a = agg_ref[0] + agg_ref[1]
  out1 = a * rdeg_ref[...] + bl1[...] + R1_ref[...]
  nrm = jnp.sqrt(jnp.sum(out1 * out1, axis=1, keepdims=True))
  out1 = out1 / jnp.maximum(nrm, 1e-12)
  h = jnp.maximum(out1 + temb_ref[...], 0.0)
  q = q_ref[...]
  f = _elu(_dot(h, Wf1h[...]) + _dot(q, Wf1q[...]) + bf1[...])
  out_ref[...] = _dot(f, Wf2[...]) + bf2[...]


def _row_spec(d):
  return pl.BlockSpec((RB, d), lambda i: (i, 0))


def _full_spec(shape):
  nd = len(shape)
  return pl.BlockSpec(shape, lambda i, _n=nd: (0,) * _n)


def _agg_spec(d):
  return pl.BlockSpec((NC, RB, d), lambda i: (0, i, 0))


def kernel(x, q_Y_sample, adj, t, num_steps, W_t1, b_t1, W_t2, b_t2,
           Wl0, bl0, Wr0, Wl1, bl1, Wr1, Wf1, bf1, Wf2, bf2):
  del num_steps
  adj = adj.astype(jnp.int32)
  src = adj[0].reshape(E // CH, CH)
  dst = adj[1].reshape(E // CH, CH)
  f32 = jnp.float32
  half = NHID // 2
  freq = jnp.exp(
      jnp.arange(half, dtype=f32) * (-math.log(10000.0) / (half - 1)))
  freq = freq.reshape(1, half)

  nfeat = x.shape[1]
  q = q_Y_sample

  ka = pl.pallas_call(
      _ka_body,
      grid=(GRID,),
      in_specs=[
          _row_spec(1), _row_spec(nfeat), _row_spec(q.shape[1]),
          _full_spec(freq.shape),
          _full_spec(W_t1.shape), _full_spec((1, b_t1.shape[0])),
          _full_spec(W_t2.shape), _full_spec((1, b_t2.shape[0])),
          _full_spec((nfeat, NHID)), _full_spec((q.shape[1], NHID)),
          _full_spec((nfeat, NHID)), _full_spec((q.shape[1], NHID)),
      ],
      out_specs=[_row_spec(80), _row_spec(NHID), _row_spec(NHID)],
      out_shape=[
          jax.ShapeDtypeStruct((N, 80), f32),
          jax.ShapeDtypeStruct((N, NHID), f32),
          jax.ShapeDtypeStruct((N, NHID), f32),
      ],
  )
  T0, R0, temb = ka(
      t.reshape(N, 1), x, q, freq,
      W_t1, b_t1.reshape(1, -1), W_t2, b_t2.reshape(1, -1),
      Wl0[:nfeat], Wl0[nfeat:], Wr0[:nfeat], Wr0[nfeat:])

  agg0 = _sc_agg80(T0, src, dst)

  kb = pl.pallas_call(
      _kb_body,
      grid=(GRID,),
      in_specs=[
          _agg_spec(80), _row_spec(NHID), _row_spec(NHID),
          _row_spec(q.shape[1]), _full_spec((1, NHID)),
          _full_spec((NHID, NHID)), _full_spec((q.shape[1], NHID)),
          _full_spec((NHID, NHID)), _full_spec((q.shape[1], NHID)),
      ],
      out_specs=[_row_spec(NHID), _row_spec(NHID), _row_spec(1)],
      out_shape=[
          jax.ShapeDtypeStruct((N, NHID), f32),
          jax.ShapeDtypeStruct((N, NHID), f32),
          jax.ShapeDtypeStruct((N, 1), f32),
      ],
  )
  T1, R1, rdeg = kb(
      agg0, R0, temb, q, bl0.reshape(1, -1),
      Wl1[:NHID], Wl1[NHID:], Wr1[:NHID], Wr1[NHID:])

  agg1 = _sc_agg64(T1, src, dst)

  nout = bf2.shape[0]
  f1 = bf1.shape[0]
  kc = pl.pallas_call(
      _kc_body,
      grid=(GRID,),
      in_specs=[
          _agg_spec(NHID), _row_spec(1), _row_spec(NHID), _row_spec(NHID),
          _row_spec(q.shape[1]), _full_spec((1, NHID)),
          _full_spec((NHID, f1)), _full_spec((q.shape[1], f1)),
          _full_spec((1, f1)), _full_spec((f1, nout)), _full_spec((1, nout)),
      ],
      out_specs=[_row_spec(nout)],
      out_shape=[jax.ShapeDtypeStruct((N, nout), f32)],
  )
  (out,) = kc(
      agg1, rdeg, R1, temb, q, bl1.reshape(1, -1),
      Wf1[:NHID], Wf1[NHID:], bf1.reshape(1, -1), Wf2, bf2.reshape(1, -1))
  return out


# speedup vs baseline: 17.8338x; 1.6820x over previous
import math

import jax
import jax.numpy as jnp
from jax import lax
from jax.experimental import pallas as pl
from jax.experimental.pallas import tpu as pltpu
from jax.experimental.pallas import tpu_sc as plsc

N = 10000
E = 320000
NHID = 64

NC = 2
NS = 16
NW = NC * NS
EPT = E // NW
CH = 80
NITER = EPT // CH
RPT = 632
NP = NS * RPT

def _dot(a, b):
  return jax.lax.dot(a, b)


def _elu(v):
  return jnp.where(v > 0, v, jnp.exp(jnp.minimum(v, 0.0)) - 1.0)


ZR = RPT // 8
NBUF = 4


def _make_sc_agg(d):
  mesh = plsc.VectorSubcoreMesh(
      core_axis_name="c", subcore_axis_name="s", num_cores=NC, num_subcores=NS)

  def body(table, adj2d, out, srcv, dstv, rows, zbuf, acc, gsem, ssem):
    c = lax.axis_index("c")
    s = lax.axis_index("s")
    wid = c * NS + s

    zeros = jnp.zeros((16,), jnp.float32)

    def zinit(r, carry):
      for k in range(d // 16):
        zbuf[r, pl.ds(k * 16, 16)] = zeros
      return carry

    lax.fori_loop(0, ZR, zinit, 0)
    for z in range(RPT // ZR):
      pltpu.sync_copy(zbuf, acc.at[pl.ds(s * RPT + z * ZR, ZR)])
    plsc.subcore_barrier()

    pltpu.sync_copy(adj2d.at[pl.ds(wid * NITER, NITER)], srcv)
    pltpu.sync_copy(adj2d.at[pl.ds(E // CH + wid * NITER, NITER)], dstv)

    for j in range(NBUF - 1):
      pltpu.async_copy(table.at[srcv.at[j]], rows.at[j], gsem.at[j])

    def eloop(i, carry):
      b = lax.rem(i, NBUF)
      pb = lax.rem(i + NBUF - 1, NBUF)
      g = i + NBUF - 1

      @pl.when(jnp.logical_and(i >= 1, g < NITER))
      def _():
        pltpu.make_async_copy(
            rows.at[pb], acc.at[dstv.at[i - 1]], ssem.at[pb]).wait()

      @pl.when(g < NITER)
      def _():
        pltpu.async_copy(table.at[srcv.at[g]], rows.at[pb], gsem.at[pb])

      pltpu.make_async_copy(table.at[srcv.at[i]], rows.at[b], gsem.at[b]).wait()
      pltpu.async_copy(rows.at[b], acc.at[dstv.at[i]], ssem.at[b], add=True)
      return carry

    lax.fori_loop(0, NITER, eloop, 0)
    for k in range(NBUF):
      ci = NITER - NBUF + k
      pltpu.make_async_copy(
          rows.at[ci % NBUF], acc.at[dstv.at[ci]], ssem.at[ci % NBUF]).wait()
    plsc.subcore_barrier()

    pltpu.sync_copy(acc.at[pl.ds(s * RPT, RPT)], out.at[c, pl.ds(s * RPT, RPT)])

  return pl.kernel(
      body,
      out_type=jax.ShapeDtypeStruct((NC, NP, d), jnp.float32),
      mesh=mesh,
      scratch_types=[
          pltpu.VMEM((NITER, CH), jnp.int32),
          pltpu.VMEM((NITER, CH), jnp.int32),
          pltpu.VMEM((NBUF, CH, d), jnp.float32),
          pltpu.VMEM((ZR, d), jnp.float32),
          pltpu.VMEM_SHARED((NP, d), jnp.float32),
          pltpu.SemaphoreType.DMA((NBUF,)),
          pltpu.SemaphoreType.DMA((NBUF,)),
      ],
      compiler_params=pltpu.CompilerParams(use_tc_tiling_on_sc=False),
  )


_sc_agg80 = _make_sc_agg(80)
_sc_agg64 = _make_sc_agg(64)


RB = 2000
GRID = N // RB


def _ka1_body(x_ref, q_ref, Wl0x, Wl0q, T0_ref):
  g = _dot(x_ref[...], Wl0x[...]) + _dot(q_ref[...], Wl0q[...])
  T0_ref[...] = jnp.concatenate(
      [g, jnp.ones((RB, 1), jnp.float32), jnp.zeros((RB, 15), jnp.float32)],
      axis=1)


def _ka2_body(t_ref, x_ref, q_ref, freq_ref, Wt1, bt1, Wt2, bt2,
              Wr0x, Wr0q, R0_ref, temb_ref):
  emb = (t_ref[...] * 4.0) * freq_ref[...]
  temb0 = jnp.concatenate([jnp.sin(emb), jnp.cos(emb)], axis=1)
  hmid = _elu(_dot(temb0, Wt1[...]) + bt1[...])
  temb_ref[...] = _dot(hmid, Wt2[...]) + bt2[...]
  R0_ref[...] = _dot(x_ref[...], Wr0x[...]) + _dot(q_ref[...], Wr0q[...])


def _kb_body(agg_ref, R0_ref, temb_ref, q_ref, bl0,
             Wl1x, Wl1q, Wr1x, Wr1q, T1_ref, R1_ref, rdeg_ref):
  a = agg_ref[0] + agg_ref[1]
  rdeg = 1.0 / jnp.maximum(a[:, 64:65], 1.0)
  out0 = a[:, :64] * rdeg + bl0[...] + R0_ref[...]
  nrm = jnp.sqrt(jnp.sum(out0 * out0, axis=1, keepdims=True))
  out0 = out0 / jnp.maximum(nrm, 1e-12)
  h = jnp.maximum(out0 + temb_ref[...], 0.0)
  q = q_ref[...]
  T1_ref[...] = _dot(h, Wl1x[...]) + _dot(q, Wl1q[...])
  R1_ref[...] = _dot(h, Wr1x[...]) + _dot(q, Wr1q[...])
  rdeg_ref[...] = rdeg


def _kc_body(agg_ref, rdeg_ref, R1_ref, temb_ref, q_ref, bl1,
             Wf1h, Wf1q, bf1, Wf2, bf2, out_ref):
  a = agg_ref[0] + agg_ref[1]
  out1 = a * rdeg_ref[...] + bl1[...] + R1_ref[...]
  nrm = jnp.sqrt(jnp.sum(out1 * out1, axis=1, keepdims=True))
  out1 = out1 / jnp.maximum(nrm, 1e-12)
  h = jnp.maximum(out1 + temb_ref[...], 0.0)
  q = q_ref[...]
  f = _elu(_dot(h, Wf1h[...]) + _dot(q, Wf1q[...]) + bf1[...])
  out_ref[...] = _dot(f, Wf2[...]) + bf2[...]


def _row_spec(d):
  return pl.BlockSpec((RB, d), lambda i: (i, 0))


def _full_spec(shape):
  nd = len(shape)
  return pl.BlockSpec(shape, lambda i, _n=nd: (0,) * _n)


def _agg_spec(d):
  return pl.BlockSpec((NC, RB, d), lambda i: (0, i, 0))


def kernel(x, q_Y_sample, adj, t, num_steps, W_t1, b_t1, W_t2, b_t2,
           Wl0, bl0, Wr0, Wl1, bl1, Wr1, Wf1, bf1, Wf2, bf2):
  del num_steps
  adj2d = adj.astype(jnp.int32).reshape(2 * (E // CH), CH)
  f32 = jnp.float32
  half = NHID // 2
  freq = jnp.exp(
      jnp.arange(half, dtype=f32) * (-math.log(10000.0) / (half - 1)))
  freq = freq.reshape(1, half)

  nfeat = x.shape[1]
  q = q_Y_sample

  ka1 = pl.pallas_call(
      _ka1_body,
      grid=(GRID,),
      in_specs=[
          _row_spec(nfeat), _row_spec(q.shape[1]),
          _full_spec((nfeat, NHID)), _full_spec((q.shape[1], NHID)),
      ],
      out_specs=[_row_spec(80)],
      out_shape=[jax.ShapeDtypeStruct((N, 80), f32)],
  )
  (T0,) = ka1(x, q, Wl0[:nfeat], Wl0[nfeat:])

  agg0 = _sc_agg80(T0, adj2d)

  ka2 = pl.pallas_call(
      _ka2_body,
      grid=(GRID,),
      in_specs=[
          _row_spec(1), _row_spec(nfeat), _row_spec(q.shape[1]),
          _full_spec(freq.shape),
          _full_spec(W_t1.shape), _full_spec((1, b_t1.shape[0])),
          _full_spec(W_t2.shape), _full_spec((1, b_t2.shape[0])),
          _full_spec((nfeat, NHID)), _full_spec((q.shape[1], NHID)),
      ],
      out_specs=[_row_spec(NHID), _row_spec(NHID)],
      out_shape=[
          jax.ShapeDtypeStruct((N, NHID), f32),
          jax.ShapeDtypeStruct((N, NHID), f32),
      ],
  )
  R0, temb = ka2(
      t.reshape(N, 1), x, q, freq,
      W_t1, b_t1.reshape(1, -1), W_t2, b_t2.reshape(1, -1),
      Wr0[:nfeat], Wr0[nfeat:])

  kb = pl.pallas_call(
      _kb_body,
      grid=(GRID,),
      in_specs=[
          _agg_spec(80), _row_spec(NHID), _row_spec(NHID),
          _row_spec(q.shape[1]), _full_spec((1, NHID)),
          _full_spec((NHID, NHID)), _full_spec((q.shape[1], NHID)),
          _full_spec((NHID, NHID)), _full_spec((q.shape[1], NHID)),
      ],
      out_specs=[_row_spec(NHID), _row_spec(NHID), _row_spec(1)],
      out_shape=[
          jax.ShapeDtypeStruct((N, NHID), f32),
          jax.ShapeDtypeStruct((N, NHID), f32),
          jax.ShapeDtypeStruct((N, 1), f32),
      ],
  )
  T1, R1, rdeg = kb(
      agg0, R0, temb, q, bl0.reshape(1, -1),
      Wl1[:NHID], Wl1[NHID:], Wr1[:NHID], Wr1[NHID:])

  agg1 = _sc_agg64(T1, adj2d)

  nout = bf2.shape[0]
  f1 = bf1.shape[0]
  kc = pl.pallas_call(
      _kc_body,
      grid=(GRID,),
      in_specs=[
          _agg_spec(NHID), _row_spec(1), _row_spec(NHID), _row_spec(NHID),
          _row_spec(q.shape[1]), _full_spec((1, NHID)),
          _full_spec((NHID, f1)), _full_spec((q.shape[1], f1)),
          _full_spec((1, f1)), _full_spec((f1, nout)), _full_spec((1, nout)),
      ],
      out_specs=[_row_spec(nout)],
      out_shape=[jax.ShapeDtypeStruct((N, nout), f32)],
  )
  (out,) = kc(
      agg1, rdeg, R1, temb, q, bl1.reshape(1, -1),
      Wf1[:NHID], Wf1[NHID:], bf1.reshape(1, -1), Wf2, bf2.reshape(1, -1))
  return out


# speedup vs baseline: 19.1069x; 1.0714x over previous
import math

import jax
import jax.numpy as jnp
from jax import lax
from jax.experimental import pallas as pl
from jax.experimental.pallas import tpu as pltpu
from jax.experimental.pallas import tpu_sc as plsc

N = 10000
E = 320000
NHID = 64

NC = 2
NS = 16
NW = NC * NS
EPT = E // NW
CH = 80
NITER = EPT // CH
RPT = 632
NP = NS * RPT

def _dot(a, b):
  return jax.lax.dot(a, b)


def _elu(v):
  return jnp.where(v > 0, v, jnp.exp(jnp.minimum(v, 0.0)) - 1.0)


ZR = RPT // 8
NBUF = 6
D = NHID


def _make_sc_agg(with_deg):
  mesh = plsc.VectorSubcoreMesh(
      core_axis_name="c", subcore_axis_name="s", num_cores=NC, num_subcores=NS)

  def body(table, adj2d, *refs):
    if with_deg:
      out, dout, srcv, dstv, rows, zbuf, degp, acc, gsem, ssem = refs
    else:
      out, srcv, dstv, rows, zbuf, acc, gsem, ssem = refs
      degp = None
    c = lax.axis_index("c")
    s = lax.axis_index("s")
    wid = c * NS + s

    zeros = jnp.zeros((16,), jnp.float32)

    def zinit(r, carry):
      for k in range(D // 16):
        zbuf[r, pl.ds(k * 16, 16)] = zeros
      return carry

    lax.fori_loop(0, ZR, zinit, 0)
    if with_deg:
      def dzinit(r, carry):
        degp[pl.ds(r * 16, 16)] = zeros
        return carry

      lax.fori_loop(0, NP // 16, dzinit, 0)
    for z in range(RPT // ZR):
      pltpu.sync_copy(zbuf, acc.at[pl.ds(s * RPT + z * ZR, ZR)])
    plsc.subcore_barrier()

    pltpu.sync_copy(adj2d.at[pl.ds(wid * NITER, NITER)], srcv)
    pltpu.sync_copy(adj2d.at[pl.ds(E // CH + wid * NITER, NITER)], dstv)

    ones = jnp.full((16,), 1.0, jnp.float32)
    for j in range(NBUF - 1):
      pltpu.async_copy(table.at[srcv.at[j]], rows.at[j], gsem.at[j])

    def eloop(i, carry):
      b = lax.rem(i, NBUF)
      pb = lax.rem(i + NBUF - 1, NBUF)
      g = i + NBUF - 1

      @pl.when(jnp.logical_and(i >= 1, g < NITER))
      def _():
        pltpu.make_async_copy(
            rows.at[pb], acc.at[dstv.at[i - 1]], ssem.at[pb]).wait()

      @pl.when(g < NITER)
      def _():
        pltpu.async_copy(table.at[srcv.at[g]], rows.at[pb], gsem.at[pb])

      if with_deg:
        for k in range(CH // 16):
          plsc.addupdate_scatter(degp, [dstv[i, pl.ds(k * 16, 16)]], ones)

      pltpu.make_async_copy(table.at[srcv.at[i]], rows.at[b], gsem.at[b]).wait()
      pltpu.async_copy(rows.at[b], acc.at[dstv.at[i]], ssem.at[b], add=True)
      return carry

    lax.fori_loop(0, NITER, eloop, 0)
    for k in range(NBUF):
      ci = NITER - NBUF + k
      pltpu.make_async_copy(
          rows.at[ci % NBUF], acc.at[dstv.at[ci]], ssem.at[ci % NBUF]).wait()
    plsc.subcore_barrier()

    pltpu.sync_copy(acc.at[pl.ds(s * RPT, RPT)], out.at[c, pl.ds(s * RPT, RPT)])
    if with_deg:
      pltpu.sync_copy(degp, dout.at[wid])

  out_types = [jax.ShapeDtypeStruct((NC, NP, D), jnp.float32)]
  scratch = [
      pltpu.VMEM((NITER, CH), jnp.int32),
      pltpu.VMEM((NITER, CH), jnp.int32),
      pltpu.VMEM((NBUF, CH, D), jnp.float32),
      pltpu.VMEM((ZR, D), jnp.float32),
  ]
  if with_deg:
    out_types.append(jax.ShapeDtypeStruct((NW, NP), jnp.float32))
    scratch.append(pltpu.VMEM((NP,), jnp.float32))
  scratch += [
      pltpu.VMEM_SHARED((NP, D), jnp.float32),
      pltpu.SemaphoreType.DMA((NBUF,)),
      pltpu.SemaphoreType.DMA((NBUF,)),
  ]
  return pl.kernel(
      body,
      out_type=out_types,
      mesh=mesh,
      scratch_types=scratch,
      compiler_params=pltpu.CompilerParams(
          use_tc_tiling_on_sc=False, needs_layout_passes=False),
  )


_sc_agg_deg = _make_sc_agg(True)
_sc_agg = _make_sc_agg(False)


RB = 2560
GRID = (N + RB - 1) // RB


def _ka1_body(x_ref, q_ref, Wl0x, Wl0q, T0_ref):
  T0_ref[...] = _dot(x_ref[...], Wl0x[...]) + _dot(q_ref[...], Wl0q[...])


def _ka2_body(t_ref, x_ref, q_ref, freq_ref, Wt1, bt1, Wt2, bt2,
              Wr0x, Wr0q, R0_ref, temb_ref):
  emb = (t_ref[...] * 4.0) * freq_ref[...]
  temb0 = jnp.concatenate([jnp.sin(emb), jnp.cos(emb)], axis=1)
  hmid = _elu(_dot(temb0, Wt1[...]) + bt1[...])
  temb_ref[...] = _dot(hmid, Wt2[...]) + bt2[...]
  R0_ref[...] = _dot(x_ref[...], Wr0x[...]) + _dot(q_ref[...], Wr0q[...])


def _kb_body(agg_ref, deg_ref, R0_ref, temb_ref, q_ref, bl0,
             Wl1x, Wl1q, Wr1x, Wr1q, T1_ref, R1_ref, rdeg_ref):
  a = agg_ref[0] + agg_ref[1]
  deg = jnp.sum(deg_ref[...], axis=0)
  rdeg = (1.0 / jnp.maximum(deg, 1.0)).reshape(RB, 1)
  out0 = a * rdeg + bl0[...] + R0_ref[...]
  nrm = jnp.sqrt(jnp.sum(out0 * out0, axis=1, keepdims=True))
  out0 = out0 / jnp.maximum(nrm, 1e-12)
  h = jnp.maximum(out0 + temb_ref[...], 0.0)
  q = q_ref[...]
  T1_ref[...] = _dot(h, Wl1x[...]) + _dot(q, Wl1q[...])
  R1_ref[...] = _dot(h, Wr1x[...]) + _dot(q, Wr1q[...])
  rdeg_ref[...] = rdeg


def _kc_body(agg_ref, rdeg_ref, R1_ref, temb_ref, q_ref, bl1,
             Wf1h, Wf1q, bf1, Wf2, bf2, out_ref):
  a = agg_ref[0] + agg_ref[1]
  out1 = a * rdeg_ref[...] + bl1[...] + R1_ref[...]
  nrm = jnp.sqrt(jnp.sum(out1 * out1, axis=1, keepdims=True))
  out1 = out1 / jnp.maximum(nrm, 1e-12)
  h = jnp.maximum(out1 + temb_ref[...], 0.0)
  q = q_ref[...]
  f = _elu(_dot(h, Wf1h[...]) + _dot(q, Wf1q[...]) + bf1[...])
  out_ref[...] = _dot(f, Wf2[...]) + bf2[...]


def _row_spec(d):
  return pl.BlockSpec((RB, d), lambda i: (i, 0))


def _full_spec(shape):
  nd = len(shape)
  return pl.BlockSpec(shape, lambda i, _n=nd: (0,) * _n)


def _agg_spec(d):
  return pl.BlockSpec((NC, RB, d), lambda i: (0, i, 0))


def kernel(x, q_Y_sample, adj, t, num_steps, W_t1, b_t1, W_t2, b_t2,
           Wl0, bl0, Wr0, Wl1, bl1, Wr1, Wf1, bf1, Wf2, bf2):
  del num_steps
  adj2d = adj.astype(jnp.int32).reshape(2 * (E // CH), CH)
  f32 = jnp.float32
  half = NHID // 2
  freq = jnp.exp(
      jnp.arange(half, dtype=f32) * (-math.log(10000.0) / (half - 1)))
  freq = freq.reshape(1, half)

  nfeat = x.shape[1]
  q = q_Y_sample

  ka1 = pl.pallas_call(
      _ka1_body,
      grid=(GRID,),
      in_specs=[
          _row_spec(nfeat), _row_spec(q.shape[1]),
          _full_spec((nfeat, NHID)), _full_spec((q.shape[1], NHID)),
      ],
      out_specs=[_row_spec(NHID)],
      out_shape=[jax.ShapeDtypeStruct((N, NHID), f32)],
  )
  (T0,) = ka1(x, q, Wl0[:nfeat], Wl0[nfeat:])

  agg0, degp = _sc_agg_deg(T0, adj2d)

  ka2 = pl.pallas_call(
      _ka2_body,
      grid=(GRID,),
      in_specs=[
          _row_spec(1), _row_spec(nfeat), _row_spec(q.shape[1]),
          _full_spec(freq.shape),
          _full_spec(W_t1.shape), _full_spec((1, b_t1.shape[0])),
          _full_spec(W_t2.shape), _full_spec((1, b_t2.shape[0])),
          _full_spec((nfeat, NHID)), _full_spec((q.shape[1], NHID)),
      ],
      out_specs=[_row_spec(NHID), _row_spec(NHID)],
      out_shape=[
          jax.ShapeDtypeStruct((N, NHID), f32),
          jax.ShapeDtypeStruct((N, NHID), f32),
      ],
  )
  R0, temb = ka2(
      t.reshape(N, 1), x, q, freq,
      W_t1, b_t1.reshape(1, -1), W_t2, b_t2.reshape(1, -1),
      Wr0[:nfeat], Wr0[nfeat:])

  kb = pl.pallas_call(
      _kb_body,
      grid=(GRID,),
      in_specs=[
          _agg_spec(NHID), pl.BlockSpec((NW, RB), lambda i: (0, i)),
          _row_spec(NHID), _row_spec(NHID),
          _row_spec(q.shape[1]), _full_spec((1, NHID)),
          _full_spec((NHID, NHID)), _full_spec((q.shape[1], NHID)),
          _full_spec((NHID, NHID)), _full_spec((q.shape[1], NHID)),
      ],
      out_specs=[_row_spec(NHID), _row_spec(NHID), _row_spec(1)],
      out_shape=[
          jax.ShapeDtypeStruct((N, NHID), f32),
          jax.ShapeDtypeStruct((N, NHID), f32),
          jax.ShapeDtypeStruct((N, 1), f32),
      ],
  )
  T1, R1, rdeg = kb(
      agg0, degp, R0, temb, q, bl0.reshape(1, -1),
      Wl1[:NHID], Wl1[NHID:], Wr1[:NHID], Wr1[NHID:])

  agg1, = _sc_agg(T1, adj2d)

  nout = bf2.shape[0]
  f1 = bf1.shape[0]
  kc = pl.pallas_call(
      _kc_body,
      grid=(GRID,),
      in_specs=[
          _agg_spec(NHID), _row_spec(1), _row_spec(NHID), _row_spec(NHID),
          _row_spec(q.shape[1]), _full_spec((1, NHID)),
          _full_spec((NHID, f1)), _full_spec((q.shape[1], f1)),
          _full_spec((1, f1)), _full_spec((f1, nout)), _full_spec((1, nout)),
      ],
      out_specs=[_row_spec(nout)],
      out_shape=[jax.ShapeDtypeStruct((N, nout), f32)],
  )
  (out,) = kc(
      agg1, rdeg, R1, temb, q, bl1.reshape(1, -1),
      Wf1[:NHID], Wf1[NHID:], bf1.reshape(1, -1), Wf2, bf2.reshape(1, -1))
  return out


# speedup vs baseline: 19.4909x; 1.0201x over previous
import math

import jax
import jax.numpy as jnp
from jax import lax
from jax.experimental import pallas as pl
from jax.experimental.pallas import tpu as pltpu
from jax.experimental.pallas import tpu_sc as plsc

N = 10000
E = 320000
NHID = 64

NC = 2
NS = 16
NW = NC * NS
EPT = E // NW
CH = 80
NITER = EPT // CH
RPT = 632
NP = NS * RPT

def _dot(a, b):
  return jax.lax.dot(a, b)


def _elu(v):
  return jnp.where(v > 0, v, jnp.exp(jnp.minimum(v, 0.0)) - 1.0)


def _rownorm(v):
  ssq = _dot(v * v, jnp.ones((v.shape[1], 1), jnp.float32))
  return v * jax.lax.rsqrt(jnp.maximum(ssq, 1e-24))


ZR = RPT // 8
NBUF = 6
D = NHID


def _make_sc_agg(with_deg):
  mesh = plsc.VectorSubcoreMesh(
      core_axis_name="c", subcore_axis_name="s", num_cores=NC, num_subcores=NS)

  def body(table, adj2d, *refs):
    if with_deg:
      out, dout, srcv, dstv, rows, zbuf, degp, acc, gsem, ssem = refs
    else:
      out, srcv, dstv, rows, zbuf, acc, gsem, ssem = refs
      degp = None
    c = lax.axis_index("c")
    s = lax.axis_index("s")
    wid = c * NS + s

    zeros = jnp.zeros((16,), jnp.float32)

    def zinit(r, carry):
      for k in range(D // 16):
        zbuf[r, pl.ds(k * 16, 16)] = zeros
      return carry

    lax.fori_loop(0, ZR, zinit, 0)
    if with_deg:
      def dzinit(r, carry):
        degp[pl.ds(r * 16, 16)] = zeros
        return carry

      lax.fori_loop(0, NP // 16, dzinit, 0)
    for z in range(RPT // ZR):
      pltpu.sync_copy(zbuf, acc.at[pl.ds(s * RPT + z * ZR, ZR)])
    plsc.subcore_barrier()

    pltpu.sync_copy(adj2d.at[pl.ds(wid * NITER, NITER)], srcv)
    pltpu.sync_copy(adj2d.at[pl.ds(E // CH + wid * NITER, NITER)], dstv)

    ones = jnp.full((16,), 1.0, jnp.float32)
    for j in range(NBUF - 1):
      pltpu.async_copy(table.at[srcv.at[j]], rows.at[j], gsem.at[j])

    def eloop(i, carry):
      b = lax.rem(i, NBUF)
      pb = lax.rem(i + NBUF - 1, NBUF)
      g = i + NBUF - 1

      @pl.when(jnp.logical_and(i >= 1, g < NITER))
      def _():
        pltpu.make_async_copy(
            rows.at[pb], acc.at[dstv.at[i - 1]], ssem.at[pb]).wait()

      @pl.when(g < NITER)
      def _():
        pltpu.async_copy(table.at[srcv.at[g]], rows.at[pb], gsem.at[pb])

      if with_deg:
        for k in range(CH // 16):
          plsc.addupdate_scatter(degp, [dstv[i, pl.ds(k * 16, 16)]], ones)

      pltpu.make_async_copy(table.at[srcv.at[i]], rows.at[b], gsem.at[b]).wait()
      pltpu.async_copy(rows.at[b], acc.at[dstv.at[i]], ssem.at[b], add=True)
      return carry

    lax.fori_loop(0, NITER, eloop, 0)
    for k in range(NBUF):
      ci = NITER - NBUF + k
      pltpu.make_async_copy(
          rows.at[ci % NBUF], acc.at[dstv.at[ci]], ssem.at[ci % NBUF]).wait()
    plsc.subcore_barrier()

    pltpu.sync_copy(acc.at[pl.ds(s * RPT, RPT)], out.at[c, pl.ds(s * RPT, RPT)])
    if with_deg:
      pltpu.sync_copy(degp, dout.at[wid])

  out_types = [jax.ShapeDtypeStruct((NC, NP, D), jnp.float32)]
  scratch = [
      pltpu.VMEM((NITER, CH), jnp.int32),
      pltpu.VMEM((NITER, CH), jnp.int32),
      pltpu.VMEM((NBUF, CH, D), jnp.float32),
      pltpu.VMEM((ZR, D), jnp.float32),
  ]
  if with_deg:
    out_types.append(jax.ShapeDtypeStruct((NW, NP), jnp.float32))
    scratch.append(pltpu.VMEM((NP,), jnp.float32))
  scratch += [
      pltpu.VMEM_SHARED((NP, D), jnp.float32),
      pltpu.SemaphoreType.DMA((NBUF,)),
      pltpu.SemaphoreType.DMA((NBUF,)),
  ]
  return pl.kernel(
      body,
      out_type=out_types,
      mesh=mesh,
      scratch_types=scratch,
      compiler_params=pltpu.CompilerParams(
          use_tc_tiling_on_sc=False, needs_layout_passes=False),
  )


_sc_agg_deg = _make_sc_agg(True)
_sc_agg = _make_sc_agg(False)


RB = 2560
GRID = (N + RB - 1) // RB


def _ka1_body(x_ref, q_ref, Wl0x, Wl0q, T0_ref):
  T0_ref[...] = _dot(x_ref[...], Wl0x[...]) + _dot(q_ref[...], Wl0q[...])


def _ka2_body(t_ref, x_ref, q_ref, freq_ref, Wt1, bt1, Wt2, bt2,
              Wr0x, Wr0q, R0_ref, temb_ref):
  emb = (t_ref[...] * 4.0) * freq_ref[...]
  temb0 = jnp.concatenate([jnp.sin(emb), jnp.cos(emb)], axis=1)
  hmid = _elu(_dot(temb0, Wt1[...]) + bt1[...])
  temb_ref[...] = _dot(hmid, Wt2[...]) + bt2[...]
  R0_ref[...] = _dot(x_ref[...], Wr0x[...]) + _dot(q_ref[...], Wr0q[...])


def _kb_body(agg_ref, deg_ref, R0_ref, temb_ref, q_ref, bl0,
             Wl1x, Wl1q, Wr1x, Wr1q, T1_ref, R1_ref, rdeg_ref):
  a = agg_ref[0] + agg_ref[1]
  deg = jnp.sum(deg_ref[...], axis=0)
  rdeg = (1.0 / jnp.maximum(deg, 1.0)).reshape(RB, 1)
  out0 = a * rdeg + bl0[...] + R0_ref[...]
  h = jnp.maximum(_rownorm(out0) + temb_ref[...], 0.0)
  q = q_ref[...]
  T1_ref[...] = _dot(h, Wl1x[...]) + _dot(q, Wl1q[...])
  R1_ref[...] = _dot(h, Wr1x[...]) + _dot(q, Wr1q[...])
  rdeg_ref[...] = rdeg


def _kc_body(agg_ref, rdeg_ref, R1_ref, temb_ref, q_ref, bl1,
             Wf1h, Wf1q, bf1, Wf2, bf2, out_ref):
  a = agg_ref[0] + agg_ref[1]
  out1 = a * rdeg_ref[...] + bl1[...] + R1_ref[...]
  h = jnp.maximum(_rownorm(out1) + temb_ref[...], 0.0)
  q = q_ref[...]
  f = _elu(_dot(h, Wf1h[...]) + _dot(q, Wf1q[...]) + bf1[...])
  out_ref[...] = _dot(f, Wf2[...]) + bf2[...]


def _row_spec(d):
  return pl.BlockSpec((RB, d), lambda i: (i, 0))


def _full_spec(shape):
  nd = len(shape)
  return pl.BlockSpec(shape, lambda i, _n=nd: (0,) * _n)


def _agg_spec(d):
  return pl.BlockSpec((NC, RB, d), lambda i: (0, i, 0))


def kernel(x, q_Y_sample, adj, t, num_steps, W_t1, b_t1, W_t2, b_t2,
           Wl0, bl0, Wr0, Wl1, bl1, Wr1, Wf1, bf1, Wf2, bf2):
  del num_steps
  adj2d = adj.astype(jnp.int32).reshape(2 * (E // CH), CH)
  f32 = jnp.float32
  half = NHID // 2
  freq = jnp.exp(
      jnp.arange(half, dtype=f32) * (-math.log(10000.0) / (half - 1)))
  freq = freq.reshape(1, half)

  nfeat = x.shape[1]
  q = q_Y_sample

  ka1 = pl.pallas_call(
      _ka1_body,
      grid=(GRID,),
      in_specs=[
          _row_spec(nfeat), _row_spec(q.shape[1]),
          _full_spec((nfeat, NHID)), _full_spec((q.shape[1], NHID)),
      ],
      out_specs=[_row_spec(NHID)],
      out_shape=[jax.ShapeDtypeStruct((N, NHID), f32)],
  )
  (T0,) = ka1(x, q, Wl0[:nfeat], Wl0[nfeat:])

  agg0, degp = _sc_agg_deg(T0, adj2d)

  ka2 = pl.pallas_call(
      _ka2_body,
      grid=(GRID,),
      in_specs=[
          _row_spec(1), _row_spec(nfeat), _row_spec(q.shape[1]),
          _full_spec(freq.shape),
          _full_spec(W_t1.shape), _full_spec((1, b_t1.shape[0])),
          _full_spec(W_t2.shape), _full_spec((1, b_t2.shape[0])),
          _full_spec((nfeat, NHID)), _full_spec((q.shape[1], NHID)),
      ],
      out_specs=[_row_spec(NHID), _row_spec(NHID)],
      out_shape=[
          jax.ShapeDtypeStruct((N, NHID), f32),
          jax.ShapeDtypeStruct((N, NHID), f32),
      ],
  )
  R0, temb = ka2(
      t.reshape(N, 1), x, q, freq,
      W_t1, b_t1.reshape(1, -1), W_t2, b_t2.reshape(1, -1),
      Wr0[:nfeat], Wr0[nfeat:])

  kb = pl.pallas_call(
      _kb_body,
      grid=(GRID,),
      in_specs=[
          _agg_spec(NHID), pl.BlockSpec((NW, RB), lambda i: (0, i)),
          _row_spec(NHID), _row_spec(NHID),
          _row_spec(q.shape[1]), _full_spec((1, NHID)),
          _full_spec((NHID, NHID)), _full_spec((q.shape[1], NHID)),
          _full_spec((NHID, NHID)), _full_spec((q.shape[1], NHID)),
      ],
      out_specs=[_row_spec(NHID), _row_spec(NHID), _row_spec(1)],
      out_shape=[
          jax.ShapeDtypeStruct((N, NHID), f32),
          jax.ShapeDtypeStruct((N, NHID), f32),
          jax.ShapeDtypeStruct((N, 1), f32),
      ],
  )
  T1, R1, rdeg = kb(
      agg0, degp, R0, temb, q, bl0.reshape(1, -1),
      Wl1[:NHID], Wl1[NHID:], Wr1[:NHID], Wr1[NHID:])

  agg1, = _sc_agg(T1, adj2d)

  nout = bf2.shape[0]
  f1 = bf1.shape[0]
  kc = pl.pallas_call(
      _kc_body,
      grid=(GRID,),
      in_specs=[
          _agg_spec(NHID), _row_spec(1), _row_spec(NHID), _row_spec(NHID),
          _row_spec(q.shape[1]), _full_spec((1, NHID)),
          _full_spec((NHID, f1)), _full_spec((q.shape[1], f1)),
          _full_spec((1, f1)), _full_spec((f1, nout)), _full_spec((1, nout)),
      ],
      out_specs=[_row_spec(nout)],
      out_shape=[jax.ShapeDtypeStruct((N, nout), f32)],
  )
  (out,) = kc(
      agg1, rdeg, R1, temb, q, bl1.reshape(1, -1),
      Wf1[:NHID], Wf1[NHID:], bf1.reshape(1, -1), Wf2, bf2.reshape(1, -1))
  return out
